# Initial kernel scaffold; baseline (speedup 1.0000x reference)
import jax
import jax.numpy as jnp
from jax.experimental import pallas as pl


def kernel(indices, table):
    raise NotImplementedError("write your pallas kernel here")


# speedup vs baseline: 1.4771x; 1.4771x over previous
import functools

import jax
import jax.numpy as jnp
from jax import lax
from jax.experimental import pallas as pl
from jax.experimental.pallas import tpu as pltpu
from jax.experimental.pallas import tpu_sc as plsc

_NW = 32
_CHUNK = 1600


def _sc_gather(table, flat_idx):
    btot = flat_idx.shape[0]
    d = table.shape[1]
    b_per_w = btot // _NW
    n_chunks = b_per_w // _CHUNK
    mesh = plsc.VectorSubcoreMesh(core_axis_name="c", subcore_axis_name="s")

    @functools.partial(
        pl.kernel,
        mesh=mesh,
        out_type=jax.ShapeDtypeStruct((btot, d), jnp.float32),
        compiler_params=pltpu.CompilerParams(use_tc_tiling_on_sc=False),
        scratch_types=[
            pltpu.VMEM((_CHUNK,), jnp.int32),
            pltpu.VMEM((_CHUNK, d), jnp.float32),
            pltpu.SemaphoreType.DMA,
        ],
    )
    def k(table_hbm, idx_hbm, out_hbm, idx_v, rows_v, sem):
        wid = lax.axis_index("s") * 2 + lax.axis_index("c")
        base = wid * b_per_w

        def body(c, carry):
            off = base + c * _CHUNK
            pltpu.sync_copy(idx_hbm.at[pl.ds(off, _CHUNK)], idx_v)
            pltpu.async_copy(table_hbm.at[idx_v], rows_v, sem).wait()
            pltpu.sync_copy(rows_v, out_hbm.at[pl.ds(off, _CHUNK)])
            return carry

        lax.fori_loop(0, n_chunks, body, 0)

    return k(table, flat_idx)


def kernel(indices, table):
    flat_idx = indices.reshape(-1)
    out = _sc_gather(table, flat_idx)
    return out.reshape(indices.shape + (table.shape[1],))


# speedup vs baseline: 1.4980x; 1.0141x over previous
import functools

import jax
import jax.numpy as jnp
from jax import lax
from jax.experimental import pallas as pl
from jax.experimental.pallas import tpu as pltpu
from jax.experimental.pallas import tpu_sc as plsc

_NW = 32
_CHUNK = 1600
_NBUF = 2


def _sc_gather(table, flat_idx):
    btot = flat_idx.shape[0]
    d = table.shape[1]
    b_per_w = btot // _NW
    n_chunks = b_per_w // _CHUNK
    mesh = plsc.VectorSubcoreMesh(core_axis_name="c", subcore_axis_name="s")

    @functools.partial(
        pl.kernel,
        mesh=mesh,
        out_type=jax.ShapeDtypeStruct((btot, d), jnp.float32),
        compiler_params=pltpu.CompilerParams(use_tc_tiling_on_sc=False),
        scratch_types=[
            pltpu.VMEM((b_per_w,), jnp.int32),
            pltpu.VMEM((_CHUNK, d), jnp.float32),
            pltpu.VMEM((_CHUNK, d), jnp.float32),
            pltpu.SemaphoreType.DMA,
            pltpu.SemaphoreType.DMA,
            pltpu.SemaphoreType.DMA,
            pltpu.SemaphoreType.DMA,
        ],
    )
    def k(table_hbm, idx_hbm, out_hbm, idx_v, rows0, rows1, g0, g1, s0, s1):
        wid = lax.axis_index("s") * 2 + lax.axis_index("c")
        base = wid * b_per_w
        pltpu.sync_copy(idx_hbm.at[pl.ds(base, b_per_w)], idx_v)
        rows = (rows0, rows1)
        gsem = (g0, g1)
        ssem = (s0, s1)

        def gather_desc(c, b):
            src = table_hbm.at[idx_v.at[pl.ds(c * _CHUNK, _CHUNK)]]
            return pltpu.make_async_copy(src, rows[b], gsem[b])

        def store_desc(c, b):
            dst = out_hbm.at[pl.ds(base + c * _CHUNK, _CHUNK)]
            return pltpu.make_async_copy(rows[b], dst, ssem[b])

        for b in range(_NBUF):
            gather_desc(b, b).start()
        for b in range(_NBUF):
            gather_desc(b, b).wait()
            store_desc(b, b).start()

        @pl.loop(_NBUF, n_chunks, step=_NBUF)
        def body(g):
            for b in range(_NBUF):
                c = g + b
                store_desc(c - _NBUF, b).wait()
                gather_desc(c, b).start()
                gather_desc(c, b).wait()
                store_desc(c, b).start()

        for b in range(_NBUF):
            store_desc(n_chunks - _NBUF + b, b).wait()

    return k(table, flat_idx)


def kernel(indices, table):
    flat_idx = indices.reshape(-1)
    out = _sc_gather(table, flat_idx)
    return out.reshape(indices.shape + (table.shape[1],))
